# baseline (speedup 1.0000x reference)
import jax
import jax.numpy as jnp
from jax.experimental import pallas as pl
from jax.experimental.pallas import tpu as pltpu

_GAMMA = 2.0
_ALPHA = 0.25
_THR_POS = 0.05
_THR_NEG = 0.02

_B, _H, _W, _NB = 16, 128, 128, 1024
_N_PIX = _B * _H * _W
_BPS = 2
_STEPS = _B // _BPS


def _loss_kernel(cl_ref, re_ref, gr_ref, gt_ref, cf_ref, op_ref, bb_ref,
                 br_ref, gb_ref, o_total, o_rpn0, o_rpn1, o_rcnn, o_pos,
                 acc_ref):
    g = pl.program_id(0)
    f32 = jnp.float32

    @pl.when(g == 0)
    def _init():
        for i in range(8):
            acc_ref[i] = 0.0

    T = gt_ref[...].astype(f32)

    x0 = cl_ref[0::2, :]
    x1 = cl_ref[1::2, :]
    lse = jnp.maximum(x0, x1) + jnp.log1p(jnp.exp(-jnp.abs(x0 - x1)))
    sel = T >= 0.5
    xt = jnp.where(sel, x0, x1)
    logpt = xt - lse
    pt = jnp.exp(logpt)
    at = jnp.where(sel, _ALPHA, 1.0 - _ALPHA)
    om = 1.0 - pt
    rpn0_s = jnp.sum(-at * om * om * logpt)

    r0 = re_ref[0::4, :]
    r1 = re_ref[1::4, :]
    r2 = re_ref[2::4, :]
    r3 = re_ref[3::4, :]
    g0 = gr_ref[0::4, :]
    g1 = gr_ref[1::4, :]
    g2 = gr_ref[2::4, :]
    g3 = gr_ref[3::4, :]
    inter = ((jnp.minimum(r0, g0) + jnp.minimum(r2, g2))
             * (jnp.minimum(r1, g1) + jnp.minimum(r3, g3)))
    ga = (g0 + g2) * (g1 + g3)
    ra = (r0 + r2) * (r1 + r3)
    union = ga + ra - inter + 1e-7
    iou = (inter + 1.0) / (union + 1.0)
    rpn1_n = jnp.sum((1.0 - iou) * T)
    rpn1_d = jnp.sum(T)

    rowdiv = jax.lax.broadcasted_iota(jnp.int32, (_BPS * 8, 1), 0) // 8

    def gcol(c):
        v = jnp.full((_BPS * 8, 1), gb_ref[0, _BPS - 1, c], dtype=f32)
        for j in range(_BPS - 2, -1, -1):
            v = jnp.where(rowdiv == j, gb_ref[0, j, c], v)
        return v

    gx1 = gcol(0)
    gy1 = gcol(1)
    gx2 = gcol(2)
    gy2 = gcol(3)
    areaA = (jnp.maximum(gx2 - gx1, 0.0)
             * jnp.maximum(gy2 - gy1, 0.0))

    def box_iou(bref, eps):
        bx1 = bref[0::4, :]
        by1 = bref[1::4, :]
        bx2 = bref[2::4, :]
        by2 = bref[3::4, :]
        whx = jnp.maximum(jnp.minimum(bx2, gx2) - jnp.maximum(bx1, gx1), 0.0)
        why = jnp.maximum(jnp.minimum(by2, gy2) - jnp.maximum(by1, gy1), 0.0)
        inter_ = whx * why
        areaB = (jnp.maximum(bx2 - bx1, 0.0)
                 * jnp.maximum(by2 - by1, 0.0))
        union_ = areaA + areaB - inter_ + eps
        return inter_ / jnp.maximum(union_, 1e-12)

    iou_d = box_iou(br_ref, 1e-7)
    iou_bb = box_iou(bb_ref, 1.0)
    pos_d = (iou_d >= _THR_POS).astype(f32)
    neg_d = (iou_d < _THR_NEG).astype(f32)
    q_bb = (1.0 - iou_bb) * pos_d

    xop = op_ref[...]
    bce = (jnp.maximum(xop, 0.0) - xop * iou_d
           + jnp.log1p(jnp.exp(-jnp.abs(xop))))
    q_op = bce * pos_d

    Ca = cf_ref[0::2, :]
    Cb = cf_ref[1::2, :]
    lsec = jnp.maximum(Ca, Cb) + jnp.log1p(jnp.exp(-jnp.abs(Ca - Cb)))
    nl0 = lsec - Ca
    nl1 = lsec - Cb

    rcnn_s = jnp.float32(0.0)
    pn_tot = jnp.float32(0.0)
    for j in range(_BPS):
        r8 = slice(8 * j, 8 * (j + 1))
        r32 = slice(32 * j, 32 * j + 8)
        pos_j = pos_d[r8]
        neg_j = neg_d[r8]
        pn = jnp.sum(pos_j)
        nn = jnp.sum(neg_j)
        s_bb = jnp.sum(q_bb[r8])
        s_op = jnp.sum(q_op[r8])
        s_cfpos = jnp.sum(nl0[r32] * pos_j)
        s_cfnegb = jnp.sum(nl1[r32] * neg_j)
        s_cfneg = jnp.sum((nl1[32 * j + 8:32 * j + 16]
                           + nl1[32 * j + 16:32 * j + 24]
                           + nl1[32 * j + 24:32 * j + 32]) * pos_j)
        pnp = pn > 0.0
        l_op = jnp.where(pnp, s_op / jnp.maximum(pn, 1.0), 0.0)
        l_cfp = jnp.where(pnp, s_cfpos / jnp.maximum(pn, 1.0), 0.0)
        l_cfnb = jnp.where(nn > 0.0, s_cfnegb / jnp.maximum(nn, 1.0), 0.0)
        l_cfn = jnp.where(pnp, s_cfneg / jnp.maximum(3.0 * pn, 1.0), 0.0)
        l_bb = jnp.where(pnp, s_bb / jnp.maximum(pn, 1.0), 0.0)
        rcnn_s = rcnn_s + jnp.where(
            pnp, l_cfp + l_cfnb + l_cfn + l_bb + l_op, 0.0)
        pn_tot = pn_tot + pn

    acc_ref[0] = acc_ref[0] + rpn0_s
    acc_ref[1] = acc_ref[1] + rpn1_n
    acc_ref[2] = acc_ref[2] + rpn1_d
    acc_ref[3] = acc_ref[3] + rcnn_s
    acc_ref[4] = acc_ref[4] + pn_tot

    @pl.when(g == _STEPS - 1)
    def _fin():
        rpn0 = acc_ref[0] / float(_N_PIX)
        rpn1 = jnp.where(acc_ref[2] > 0.0,
                         acc_ref[1] / jnp.maximum(acc_ref[2], 1.0), 0.0)
        rcnn = acc_ref[3] / float(_B)
        o_total[0, 0] = rpn0 + rpn1 + rcnn
        o_rpn0[0, 0] = rpn0
        o_rpn1[0, 0] = rpn1
        o_rcnn[0, 0] = rcnn
        o_pos[0, 0] = acc_ref[4]


def kernel(cl, re, cf, op, bb, br, gb, gr, gt):
    clv = cl.transpose(0, 1, 3, 2).reshape(_B * 2 * _H, _W)
    rev = re.transpose(0, 1, 3, 2).reshape(_B * 4 * _H, _W)
    grv = gr.transpose(0, 1, 3, 2).reshape(_B * 4 * _H, _W)
    gtv = gt.reshape(_B * _H, _W)
    cfv = cf.reshape(_B, 8, 128, 4, 2).transpose(0, 3, 1, 4, 2) \
            .reshape(_B * 64, 128)
    opv = op.reshape(_B * 8, 128)
    bbv = bb.reshape(_B, 8, 128, 4).transpose(0, 1, 3, 2).reshape(_B * 32, 128)
    brv = br.reshape(_B, 8, 128, 4).transpose(0, 1, 3, 2).reshape(_B * 32, 128)
    gb3 = gb.reshape(_STEPS, _BPS, 4)

    scal = jax.ShapeDtypeStruct((1, 1), jnp.float32)
    outs = pl.pallas_call(
        _loss_kernel,
        grid=(_STEPS,),
        in_specs=[
            pl.BlockSpec((_BPS * 2 * _H, _W), lambda g: (g, 0)),
            pl.BlockSpec((_BPS * 4 * _H, _W), lambda g: (g, 0)),
            pl.BlockSpec((_BPS * 4 * _H, _W), lambda g: (g, 0)),
            pl.BlockSpec((_BPS * _H, _W), lambda g: (g, 0)),
            pl.BlockSpec((_BPS * 64, 128), lambda g: (g, 0)),
            pl.BlockSpec((_BPS * 8, 128), lambda g: (g, 0)),
            pl.BlockSpec((_BPS * 32, 128), lambda g: (g, 0)),
            pl.BlockSpec((_BPS * 32, 128), lambda g: (g, 0)),
            pl.BlockSpec((1, _BPS, 4), lambda g: (g, 0, 0),
                         memory_space=pltpu.SMEM),
        ],
        out_specs=[pl.BlockSpec((1, 1), lambda g: (0, 0),
                                memory_space=pltpu.SMEM)] * 5,
        out_shape=[scal] * 5,
        scratch_shapes=[pltpu.SMEM((8,), jnp.float32)],
        compiler_params=pltpu.CompilerParams(
            dimension_semantics=("arbitrary",)),
    )(clv, rev, grv, gtv, cfv, opv, bbv, brv, gb3)

    return tuple(o[0, 0] for o in outs)


# speedup vs baseline: 1.1047x; 1.1047x over previous
import jax
import jax.numpy as jnp
from jax.experimental import pallas as pl
from jax.experimental.pallas import tpu as pltpu

_GAMMA = 2.0
_ALPHA = 0.25
_THR_POS = 0.05
_THR_NEG = 0.02

_B, _H, _W, _NB = 16, 128, 128, 1024
_N_PIX = _B * _H * _W
_BPS = 4
_STEPS = _B // _BPS


def _loss_kernel(cl_ref, re_ref, gr_ref, gt_ref, cf_ref, op_ref, bb_ref,
                 br_ref, gb_ref, o_total, o_rpn0, o_rpn1, o_rcnn, o_pos,
                 acc_ref):
    g = pl.program_id(0)
    f32 = jnp.float32

    @pl.when(g == 0)
    def _init():
        for i in range(8):
            acc_ref[i] = 0.0

    T = gt_ref[...].astype(f32)

    x0 = cl_ref[0::2, :]
    x1 = cl_ref[1::2, :]
    lse = jnp.maximum(x0, x1) + jnp.log1p(jnp.exp(-jnp.abs(x0 - x1)))
    sel = T >= 0.5
    xt = jnp.where(sel, x0, x1)
    logpt = xt - lse
    pt = jnp.exp(logpt)
    at = jnp.where(sel, _ALPHA, 1.0 - _ALPHA)
    om = 1.0 - pt
    rpn0_s = jnp.sum(-at * om * om * logpt)

    r0 = re_ref[0::4, :]
    r1 = re_ref[1::4, :]
    r2 = re_ref[2::4, :]
    r3 = re_ref[3::4, :]
    g0 = gr_ref[0::4, :]
    g1 = gr_ref[1::4, :]
    g2 = gr_ref[2::4, :]
    g3 = gr_ref[3::4, :]
    inter = ((jnp.minimum(r0, g0) + jnp.minimum(r2, g2))
             * (jnp.minimum(r1, g1) + jnp.minimum(r3, g3)))
    ga = (g0 + g2) * (g1 + g3)
    ra = (r0 + r2) * (r1 + r3)
    union = ga + ra - inter + 1e-7
    iou = (inter + 1.0) / (union + 1.0)
    rpn1_n = jnp.sum((1.0 - iou) * T)
    rpn1_d = jnp.sum(T)

    rowdiv = jax.lax.broadcasted_iota(jnp.int32, (_BPS * 8, 1), 0) // 8

    def gcol(c):
        v = jnp.full((_BPS * 8, 1), gb_ref[0, _BPS - 1, c], dtype=f32)
        for j in range(_BPS - 2, -1, -1):
            v = jnp.where(rowdiv == j, gb_ref[0, j, c], v)
        return v

    gx1 = gcol(0)
    gy1 = gcol(1)
    gx2 = gcol(2)
    gy2 = gcol(3)
    areaA = (jnp.maximum(gx2 - gx1, 0.0)
             * jnp.maximum(gy2 - gy1, 0.0))

    def box_iou(bref, eps):
        bx1 = bref[0::4, :]
        by1 = bref[1::4, :]
        bx2 = bref[2::4, :]
        by2 = bref[3::4, :]
        whx = jnp.maximum(jnp.minimum(bx2, gx2) - jnp.maximum(bx1, gx1), 0.0)
        why = jnp.maximum(jnp.minimum(by2, gy2) - jnp.maximum(by1, gy1), 0.0)
        inter_ = whx * why
        areaB = (jnp.maximum(bx2 - bx1, 0.0)
                 * jnp.maximum(by2 - by1, 0.0))
        union_ = areaA + areaB - inter_ + eps
        return inter_ / jnp.maximum(union_, 1e-12)

    iou_d = box_iou(br_ref, 1e-7)
    iou_bb = box_iou(bb_ref, 1.0)
    pos_d = (iou_d >= _THR_POS).astype(f32)
    neg_d = (iou_d < _THR_NEG).astype(f32)
    q_bb = (1.0 - iou_bb) * pos_d

    xop = op_ref[...]
    bce = (jnp.maximum(xop, 0.0) - xop * iou_d
           + jnp.log1p(jnp.exp(-jnp.abs(xop))))
    q_op = bce * pos_d

    Ca = cf_ref[0::2, :]
    Cb = cf_ref[1::2, :]
    lsec = jnp.maximum(Ca, Cb) + jnp.log1p(jnp.exp(-jnp.abs(Ca - Cb)))
    nl0 = lsec - Ca
    nl1 = lsec - Cb

    rcnn_s = jnp.float32(0.0)
    pn_tot = jnp.float32(0.0)
    for j in range(_BPS):
        r8 = slice(8 * j, 8 * (j + 1))
        r32 = slice(32 * j, 32 * j + 8)
        pos_j = pos_d[r8]
        neg_j = neg_d[r8]
        pn = jnp.sum(pos_j)
        nn = jnp.sum(neg_j)
        s_bb = jnp.sum(q_bb[r8])
        s_op = jnp.sum(q_op[r8])
        s_cfpos = jnp.sum(nl0[r32] * pos_j)
        s_cfnegb = jnp.sum(nl1[r32] * neg_j)
        s_cfneg = jnp.sum((nl1[32 * j + 8:32 * j + 16]
                           + nl1[32 * j + 16:32 * j + 24]
                           + nl1[32 * j + 24:32 * j + 32]) * pos_j)
        pnp = pn > 0.0
        l_op = jnp.where(pnp, s_op / jnp.maximum(pn, 1.0), 0.0)
        l_cfp = jnp.where(pnp, s_cfpos / jnp.maximum(pn, 1.0), 0.0)
        l_cfnb = jnp.where(nn > 0.0, s_cfnegb / jnp.maximum(nn, 1.0), 0.0)
        l_cfn = jnp.where(pnp, s_cfneg / jnp.maximum(3.0 * pn, 1.0), 0.0)
        l_bb = jnp.where(pnp, s_bb / jnp.maximum(pn, 1.0), 0.0)
        rcnn_s = rcnn_s + jnp.where(
            pnp, l_cfp + l_cfnb + l_cfn + l_bb + l_op, 0.0)
        pn_tot = pn_tot + pn

    acc_ref[0] = acc_ref[0] + rpn0_s
    acc_ref[1] = acc_ref[1] + rpn1_n
    acc_ref[2] = acc_ref[2] + rpn1_d
    acc_ref[3] = acc_ref[3] + rcnn_s
    acc_ref[4] = acc_ref[4] + pn_tot

    @pl.when(g == _STEPS - 1)
    def _fin():
        rpn0 = acc_ref[0] / float(_N_PIX)
        rpn1 = jnp.where(acc_ref[2] > 0.0,
                         acc_ref[1] / jnp.maximum(acc_ref[2], 1.0), 0.0)
        rcnn = acc_ref[3] / float(_B)
        o_total[0, 0] = rpn0 + rpn1 + rcnn
        o_rpn0[0, 0] = rpn0
        o_rpn1[0, 0] = rpn1
        o_rcnn[0, 0] = rcnn
        o_pos[0, 0] = acc_ref[4]


def kernel(cl, re, cf, op, bb, br, gb, gr, gt):
    clv = cl.transpose(0, 1, 3, 2).reshape(_B * 2 * _H, _W)
    rev = re.transpose(0, 1, 3, 2).reshape(_B * 4 * _H, _W)
    grv = gr.transpose(0, 1, 3, 2).reshape(_B * 4 * _H, _W)
    gtv = gt.reshape(_B * _H, _W)
    cfv = cf.reshape(_B, 8, 128, 4, 2).transpose(0, 3, 1, 4, 2) \
            .reshape(_B * 64, 128)
    opv = op.reshape(_B * 8, 128)
    bbv = bb.reshape(_B, 8, 128, 4).transpose(0, 1, 3, 2).reshape(_B * 32, 128)
    brv = br.reshape(_B, 8, 128, 4).transpose(0, 1, 3, 2).reshape(_B * 32, 128)
    gb3 = gb.reshape(_STEPS, _BPS, 4)

    scal = jax.ShapeDtypeStruct((1, 1), jnp.float32)
    outs = pl.pallas_call(
        _loss_kernel,
        grid=(_STEPS,),
        in_specs=[
            pl.BlockSpec((_BPS * 2 * _H, _W), lambda g: (g, 0)),
            pl.BlockSpec((_BPS * 4 * _H, _W), lambda g: (g, 0)),
            pl.BlockSpec((_BPS * 4 * _H, _W), lambda g: (g, 0)),
            pl.BlockSpec((_BPS * _H, _W), lambda g: (g, 0)),
            pl.BlockSpec((_BPS * 64, 128), lambda g: (g, 0)),
            pl.BlockSpec((_BPS * 8, 128), lambda g: (g, 0)),
            pl.BlockSpec((_BPS * 32, 128), lambda g: (g, 0)),
            pl.BlockSpec((_BPS * 32, 128), lambda g: (g, 0)),
            pl.BlockSpec((1, _BPS, 4), lambda g: (g, 0, 0),
                         memory_space=pltpu.SMEM),
        ],
        out_specs=[pl.BlockSpec((1, 1), lambda g: (0, 0),
                                memory_space=pltpu.SMEM)] * 5,
        out_shape=[scal] * 5,
        scratch_shapes=[pltpu.SMEM((8,), jnp.float32)],
        compiler_params=pltpu.CompilerParams(
            dimension_semantics=("arbitrary",)),
    )(clv, rev, grv, gtv, cfv, opv, bbv, brv, gb3)

    return tuple(o[0, 0] for o in outs)
